# Initial kernel scaffold; baseline (speedup 1.0000x reference)
import jax
import jax.numpy as jnp
from jax.experimental import pallas as pl


def kernel(x, edge_index, batch, W_l, b_l, W_r):
    raise NotImplementedError("write your pallas kernel here")


# speedup vs baseline: 3.9446x; 3.9446x over previous
import functools

import jax
import jax.numpy as jnp
from jax import lax
from jax.experimental import pallas as pl
from jax.experimental.pallas import tpu as pltpu
from jax.experimental.pallas import tpu_sc as plsc

N = 10000
E = 320000
D = 128
G = 16

NC, NS = 2, 16
NW = NC * NS
C = 80
BLK = 8
NBLK = 16
EPT = NBLK * BLK * C
EPAD = NW * EPT
RPT = 624
TAIL0 = NS * RPT
N_PAD = N + 16
TAILZ = N_PAD - TAIL0
TAILC = N - TAIL0


_MESH = plsc.VectorSubcoreMesh(core_axis_name="c", subcore_axis_name="s")


@functools.partial(
    pl.kernel,
    out_type=[
        jax.ShapeDtypeStruct((NC, N, D), jnp.float32),
        jax.ShapeDtypeStruct((NW, 1, N), jnp.float32),
    ],
    mesh=_MESH,
    compiler_params=pltpu.CompilerParams(needs_layout_passes=False),
    scratch_types=[
        pltpu.VMEM((BLK, C), jnp.int32),
        pltpu.VMEM((BLK, C), jnp.int32),
        pltpu.VMEM((C, D), jnp.float32),
        pltpu.VMEM((N_PAD,), jnp.float32),
        pltpu.VMEM_SHARED((N_PAD, D), jnp.float32),
        pltpu.SemaphoreType.DMA,
    ],
)
def _sc_aggregate(x_hbm, src_hbm, dst_hbm, zrows_hbm, zdeg_hbm,
                  agg_out, deg_out, src_v, dst_v, rows_v, deg_v, acc, sem):
    cid = lax.axis_index("c")
    sid = lax.axis_index("s")
    wid = cid * NS + sid
    row0 = sid * RPT

    pltpu.sync_copy(zrows_hbm, acc.at[pl.ds(row0, RPT)])

    @pl.when(sid == 0)
    def _zero_tail():
        pltpu.sync_copy(zrows_hbm.at[pl.ds(0, TAILZ)],
                        acc.at[pl.ds(TAIL0, TAILZ)])

    pltpu.sync_copy(zdeg_hbm, deg_v)

    plsc.subcore_barrier()

    ones = jnp.ones((16,), jnp.float32)

    def body(b, carry):
        pltpu.sync_copy(src_hbm.at[wid, b], src_v)
        pltpu.sync_copy(dst_hbm.at[wid, b], dst_v)
        for j in range(BLK):
            pltpu.async_copy(x_hbm.at[src_v.at[j]], rows_v, sem).wait()
            pltpu.sync_copy(rows_v, acc.at[dst_v.at[j]], add=True)
            for k in range(C // 16):
                idx = dst_v[j, pl.ds(k * 16, 16)]
                plsc.addupdate_scatter(deg_v, [idx], ones)
        return carry

    lax.fori_loop(0, NBLK, body, 0)

    plsc.subcore_barrier()

    pltpu.sync_copy(acc.at[pl.ds(row0, RPT)], agg_out.at[cid, pl.ds(row0, RPT)])

    @pl.when(sid == 0)
    def _copy_tail():
        pltpu.sync_copy(acc.at[pl.ds(TAIL0, TAILC)],
                        agg_out.at[cid, pl.ds(TAIL0, TAILC)])

    pltpu.sync_copy(deg_v.at[pl.ds(0, N)], deg_out.at[wid, 0])


def _tc_body(agg0_ref, agg1_ref, degt_ref, x_ref, wl_ref, bl_ref, wr_ref,
             batch_ref, out_ref):
    agg = agg0_ref[...] + agg1_ref[...]
    deg = jnp.sum(degt_ref[...], axis=1, keepdims=True)
    mean_agg = agg / jnp.maximum(deg, 1.0)
    h = lax.dot_general(mean_agg, wl_ref[...], (((1,), (1,)), ((), ())),
                        preferred_element_type=jnp.float32)
    h += lax.dot_general(x_ref[...], wr_ref[...], (((1,), (1,)), ((), ())),
                         preferred_element_type=jnp.float32)
    h = jnp.tanh(h + bl_ref[...])
    onehot = (batch_ref[...] ==
              lax.broadcasted_iota(jnp.int32, (N, G), 1)).astype(jnp.float32)
    pooled = lax.dot_general(onehot, h, (((0,), (0,)), ((), ())),
                             preferred_element_type=jnp.float32)
    counts = lax.dot_general(onehot, jnp.ones((N, 1), jnp.float32),
                             (((0,), (0,)), ((), ())),
                             preferred_element_type=jnp.float32)
    out_ref[...] = pooled / jnp.maximum(counts, 1.0)


_tc_post = pl.pallas_call(
    _tc_body,
    out_shape=jax.ShapeDtypeStruct((G, D), jnp.float32),
)


def kernel(x, edge_index, batch, W_l, b_l, W_r):
    pad = EPAD - E
    src = jnp.concatenate([edge_index[0], jnp.zeros((pad,), jnp.int32)])
    dst = jnp.concatenate([edge_index[1], jnp.full((pad,), N, jnp.int32)])
    src = src.reshape(NW, NBLK, BLK, C)
    dst = dst.reshape(NW, NBLK, BLK, C)
    zrows = jnp.zeros((RPT, D), jnp.float32)
    zdeg = jnp.zeros((N_PAD,), jnp.float32)
    agg_parts, deg_parts = _sc_aggregate(x, src, dst, zrows, zdeg)
    deg_parts = deg_parts.reshape(NW, N)
    return _tc_post(agg_parts[0], agg_parts[1], deg_parts.T, x,
                    W_l, b_l.reshape(1, D), W_r, batch.reshape(N, 1))
